# baseline (speedup 1.0000x reference)
import functools

import jax
import jax.numpy as jnp
from jax import lax
from jax.experimental import pallas as pl
from jax.experimental.pallas import tpu as pltpu
from jax.experimental.pallas import tpu_sc as plsc

_NC = 2
_NS = 16
_NW = _NC * _NS
_LANES = 16


@functools.lru_cache(maxsize=None)
def _make_gather(n, npairs, width):
    assert n % _NW == 0
    bpw = n // _NW
    ch = 320
    while bpw % (2 * ch):
        ch //= 2
    npair = bpw // (2 * ch)
    nblk = ch // _LANES

    mesh = plsc.VectorSubcoreMesh(core_axis_name="c", subcore_axis_name="s")

    @functools.partial(
        pl.kernel,
        out_type=jax.ShapeDtypeStruct((n, width), jnp.float32),
        mesh=mesh,
        scratch_types=[
            pltpu.VMEM((bpw,), jnp.int32),
            pltpu.VMEM((ch,), jnp.int32),
            pltpu.VMEM((ch,), jnp.int32),
            pltpu.VMEM((ch, width), jnp.float32),
            pltpu.VMEM((ch, width), jnp.float32),
            pltpu.SemaphoreType.DMA,
            pltpu.SemaphoreType.DMA,
            pltpu.SemaphoreType.DMA,
            pltpu.SemaphoreType.DMA,
        ],
        compiler_params=pltpu.CompilerParams(needs_layout_passes=False),
    )
    def gather(t2_hbm, idx_hbm, out_hbm, idx_v, p0, p1, b0, b1,
               g0, g1, w0, w1):
        wid = lax.axis_index("s") * _NC + lax.axis_index("c")
        base = wid * bpw
        pltpu.sync_copy(idx_hbm.at[pl.ds(base, bpw)], idx_v)

        iota = lax.iota(jnp.int32, _LANES)
        half = width // 2

        def prep(c, pbuf):
            def blk(j, carry):
                vals = idx_v[pl.ds(c * ch + j * _LANES, _LANES)]
                pbuf[pl.ds(j * _LANES, _LANES)] = lax.shift_right_logical(
                    vals, 1)
                return carry
            lax.fori_loop(0, nblk, blk, 0)

        def extract(c, buf):
            def blk(j, carry):
                rows = jnp.full((_LANES,), j * _LANES, jnp.int32) + iota
                vals = idx_v[pl.ds(c * ch + j * _LANES, _LANES)]
                src_col = (vals & 1) * half
                for d in range(half):
                    e = plsc.load_gather(buf, [rows, src_col + d])
                    plsc.store_scatter(
                        buf, [rows, jnp.full((_LANES,), d, jnp.int32)], e)
                return carry
            lax.fori_loop(0, nblk, blk, 0)

        def g_copy(pbuf, buf, sem):
            return pltpu.make_async_copy(t2_hbm.at[pbuf], buf, sem)

        def w_copy(c, buf, sem):
            return pltpu.make_async_copy(
                buf, out_hbm.at[pl.ds(base + c * ch, ch)], sem)

        prep(0, p0)
        g_copy(p0, b0, g0).start()

        def body(i, carry):
            c0 = 2 * i
            g_copy(p0, b0, g0).wait()
            extract(c0, b0)
            w_copy(c0, b0, w0).start()
            pl.when(i > 0)(lambda: w_copy(c0 - 1, b1, w1).wait())
            prep(c0 + 1, p1)
            g_copy(p1, b1, g1).start()
            w_copy(c0, b0, w0).wait()

            def start_next():
                prep(c0 + 2, p0)
                g_copy(p0, b0, g0).start()
            pl.when(i + 1 < npair)(start_next)

            g_copy(p1, b1, g1).wait()
            extract(c0 + 1, b1)
            w_copy(c0 + 1, b1, w1).start()
            return carry

        lax.fori_loop(0, npair, body, 0)
        w_copy(2 * npair - 1, b1, w1).wait()

    return gather


def kernel(x, table):
    b, l = x.shape
    vocab, dim = table.shape
    t2 = table.reshape(vocab // 2, 2 * dim)
    xf = x.reshape(b * l).astype(jnp.int32)
    out2 = _make_gather(b * l, vocab // 2, 2 * dim)(t2, xf)
    return out2[:, :dim].reshape(b, l, dim)


# speedup vs baseline: 1.0455x; 1.0455x over previous
import functools

import jax
import jax.numpy as jnp
from jax import lax
from jax.experimental import pallas as pl
from jax.experimental.pallas import tpu as pltpu
from jax.experimental.pallas import tpu_sc as plsc

_NC = 2
_NS = 16
_NW = _NC * _NS
_LANES = 16


@functools.lru_cache(maxsize=None)
def _make_gather(n, npairs, width):
    assert n % _NW == 0
    bpw = n // _NW
    ch = 160
    while bpw % (2 * ch):
        ch //= 2
    npair = bpw // (2 * ch)
    nblk = ch // _LANES

    mesh = plsc.VectorSubcoreMesh(core_axis_name="c", subcore_axis_name="s")

    @functools.partial(
        pl.kernel,
        out_type=jax.ShapeDtypeStruct((n, width), jnp.float32),
        mesh=mesh,
        scratch_types=[
            pltpu.VMEM((bpw,), jnp.int32),
            pltpu.VMEM((ch,), jnp.int32),
            pltpu.VMEM((ch,), jnp.int32),
            pltpu.VMEM((ch, width), jnp.float32),
            pltpu.VMEM((ch, width), jnp.float32),
            pltpu.VMEM((ch, width), jnp.float32),
            pltpu.VMEM((ch, width), jnp.float32),
            pltpu.SemaphoreType.DMA,
            pltpu.SemaphoreType.DMA,
            pltpu.SemaphoreType.DMA,
            pltpu.SemaphoreType.DMA,
        ],
        compiler_params=pltpu.CompilerParams(needs_layout_passes=False),
    )
    def gather(t2_hbm, idx_hbm, out_hbm, idx_v, p0, p1, b0, b1, d0, d1,
               g0, g1, w0, w1):
        wid = lax.axis_index("s") * _NC + lax.axis_index("c")
        base = wid * bpw
        pltpu.sync_copy(idx_hbm.at[pl.ds(base, bpw)], idx_v)

        iota = lax.iota(jnp.int32, _LANES)
        half = width // 2

        def prep(c, pbuf):
            def blk(j, carry):
                vals = idx_v[pl.ds(c * ch + j * _LANES, _LANES)]
                pbuf[pl.ds(j * _LANES, _LANES)] = lax.shift_right_logical(
                    vals, 1)
                return carry
            lax.fori_loop(0, nblk, blk, 0)

        def extract(c, buf, dbuf):
            def blk(j, carry):
                rows = jnp.full((_LANES,), j * _LANES, jnp.int32) + iota
                vals = idx_v[pl.ds(c * ch + j * _LANES, _LANES)]
                src_col = (vals & 1) * half
                for d in range(half):
                    e = plsc.load_gather(buf, [rows, src_col + d])
                    plsc.store_scatter(
                        dbuf, [rows, jnp.full((_LANES,), d, jnp.int32)], e)
                return carry
            lax.fori_loop(0, nblk, blk, 0)

        def g_copy(pbuf, buf, sem):
            return pltpu.make_async_copy(t2_hbm.at[pbuf], buf, sem)

        def w_copy(c, dbuf, sem):
            return pltpu.make_async_copy(
                dbuf, out_hbm.at[pl.ds(base + c * ch, ch)], sem)

        prep(0, p0)
        g_copy(p0, b0, g0).start()

        def body(i, carry):
            c0 = 2 * i
            g_copy(p0, b0, g0).wait()
            prep(c0 + 1, p1)
            g_copy(p1, b1, g1).start()
            extract(c0, b0, d0)
            pl.when(i > 0)(lambda: w_copy(c0 - 1, d1, w1).wait())
            w_copy(c0, d0, w0).start()

            def start_next():
                prep(c0 + 2, p0)
                g_copy(p0, b0, g0).start()
            pl.when(i + 1 < npair)(start_next)

            g_copy(p1, b1, g1).wait()
            extract(c0 + 1, b1, d1)
            w_copy(c0, d0, w0).wait()
            w_copy(c0 + 1, d1, w1).start()
            return carry

        lax.fori_loop(0, npair, body, 0)
        w_copy(2 * npair - 1, d1, w1).wait()

    return gather


def kernel(x, table):
    b, l = x.shape
    vocab, dim = table.shape
    t2 = table.reshape(vocab // 2, 2 * dim)
    xf = x.reshape(b * l).astype(jnp.int32)
    out2 = _make_gather(b * l, vocab // 2, 2 * dim)(t2, xf)
    return out2[:, :dim].reshape(b, l, dim)


# speedup vs baseline: 2.5822x; 2.4698x over previous
import functools

import jax
import jax.numpy as jnp
from jax import lax
from jax.experimental import pallas as pl
from jax.experimental.pallas import tpu as pltpu
from jax.experimental.pallas import tpu_sc as plsc

_NC = 2
_NS = 16
_NW = _NC * _NS
_LANES = 16


@functools.lru_cache(maxsize=None)
def _make_gather(n, npairs, width):
    assert n % _NW == 0
    bpw = n // _NW
    ch = 160
    while bpw % (2 * ch):
        ch //= 2
    npair = bpw // (2 * ch)
    nblk = ch // _LANES

    mesh = plsc.VectorSubcoreMesh(core_axis_name="c", subcore_axis_name="s")

    @functools.partial(
        pl.kernel,
        out_type=jax.ShapeDtypeStruct((n, width), jnp.float32),
        mesh=mesh,
        scratch_types=[
            pltpu.VMEM((bpw,), jnp.int32),
            pltpu.VMEM((ch,), jnp.int32),
            pltpu.VMEM((ch,), jnp.int32),
            pltpu.VMEM((ch, width), jnp.float32),
            pltpu.VMEM((ch, width), jnp.float32),
            pltpu.VMEM((ch, width), jnp.float32),
            pltpu.VMEM((ch, width), jnp.float32),
            pltpu.SemaphoreType.DMA,
            pltpu.SemaphoreType.DMA,
            pltpu.SemaphoreType.DMA,
            pltpu.SemaphoreType.DMA,
        ],
        compiler_params=pltpu.CompilerParams(needs_layout_passes=False),
    )
    def gather(t2_hbm, idx_hbm, out_hbm, idx_v, p0, p1, b0, b1, d0, d1,
               g0, g1, w0, w1):
        wid = lax.axis_index("s") * _NC + lax.axis_index("c")
        base = wid * bpw
        pltpu.sync_copy(idx_hbm.at[pl.ds(base, bpw)], idx_v)

        iota = lax.iota(jnp.int32, _LANES)
        half = width // 2

        def prep(c, pbuf):
            def blk(j, carry):
                vals = idx_v[pl.ds(c * ch + j * _LANES, _LANES)]
                pbuf[pl.ds(j * _LANES, _LANES)] = lax.shift_right_logical(
                    vals, 1)
                return carry
            lax.fori_loop(0, nblk, blk, 0)

        def extract(c, buf, dbuf):
            @functools.partial(plsc.parallel_loop, 0, nblk)
            def blk(j):
                rows = jnp.full((_LANES,), j * _LANES, jnp.int32) + iota
                vals = idx_v[pl.ds(c * ch + j * _LANES, _LANES)]
                src_col = (vals & 1) * half
                grp = 8
                for d0 in range(0, half, grp):
                    es = [plsc.load_gather(buf, [rows, src_col + (d0 + k)])
                          for k in range(grp)]
                    for k in range(grp):
                        plsc.store_scatter(
                            dbuf,
                            [rows, jnp.full((_LANES,), d0 + k, jnp.int32)],
                            es[k])

        def g_copy(pbuf, buf, sem):
            return pltpu.make_async_copy(t2_hbm.at[pbuf], buf, sem)

        def w_copy(c, dbuf, sem):
            return pltpu.make_async_copy(
                dbuf, out_hbm.at[pl.ds(base + c * ch, ch)], sem)

        prep(0, p0)
        g_copy(p0, b0, g0).start()

        def body(i, carry):
            c0 = 2 * i
            g_copy(p0, b0, g0).wait()
            prep(c0 + 1, p1)
            g_copy(p1, b1, g1).start()
            extract(c0, b0, d0)
            pl.when(i > 0)(lambda: w_copy(c0 - 1, d1, w1).wait())
            w_copy(c0, d0, w0).start()

            def start_next():
                prep(c0 + 2, p0)
                g_copy(p0, b0, g0).start()
            pl.when(i + 1 < npair)(start_next)

            g_copy(p1, b1, g1).wait()
            extract(c0 + 1, b1, d1)
            w_copy(c0, d0, w0).wait()
            w_copy(c0 + 1, d1, w1).start()
            return carry

        lax.fori_loop(0, npair, body, 0)
        w_copy(2 * npair - 1, d1, w1).wait()

    return gather


def kernel(x, table):
    b, l = x.shape
    vocab, dim = table.shape
    t2 = table.reshape(vocab // 2, 2 * dim)
    xf = x.reshape(b * l).astype(jnp.int32)
    out2 = _make_gather(b * l, vocab // 2, 2 * dim)(t2, xf)
    return out2[:, :dim].reshape(b, l, dim)
